# baseline (speedup 1.0000x reference)
import functools

import jax
import jax.numpy as jnp
from jax import lax
from jax.experimental import pallas as pl
from jax.experimental.pallas import tpu as pltpu
from jax.experimental.pallas import tpu_sc as plsc

B = 16384
L = 200
V = 17
D = 16
LP = 256
NC, NS = 2, 16
NW = NC * NS
XR = B // 8
XC = 8 * L // 4
WROWS = XR // NW
CROWS = 32
NCHUNK = WROWS // CROWS
NSLICE = XC // 16

_mesh = plsc.VectorSubcoreMesh(core_axis_name="c", subcore_axis_name="s")


@functools.partial(
    pl.kernel,
    out_type=jax.ShapeDtypeStruct((NW, V, LP), jnp.float32),
    mesh=_mesh,
    compiler_params=pltpu.CompilerParams(needs_layout_passes=False),
    scratch_types=[
        pltpu.VMEM((CROWS, XC), jnp.int32),
        pltpu.VMEM((CROWS, XC), jnp.int32),
        pltpu.VMEM((V, LP), jnp.float32),
        pltpu.SemaphoreType.DMA,
        pltpu.SemaphoreType.DMA,
    ],
)
def _sc_hist(x_hbm, out_hbm, xb0, xb1, cnt, sem0, sem1):
    wid = lax.axis_index("s") * NC + lax.axis_index("c")
    bufs = (xb0, xb1)
    sems = (sem0, sem1)

    @plsc.parallel_loop(0, V, unroll=1)
    def _(j):
        for s in range(LP // 16):
            cnt[j, pl.ds(s * 16, 16)] = jnp.zeros((16,), jnp.float32)

    row0 = wid * WROWS

    def start(k):
        return pltpu.async_copy(
            x_hbm.at[pl.ds(row0 + k * CROWS, CROWS)], bufs[k % 2], sems[k % 2])

    ones = jnp.ones((16,), jnp.float32)
    iota = lax.iota(jnp.int32, 16)
    lvecs = [jnp.remainder(iota * 4 + s * 64, L) for s in range(NSLICE)]

    descs = [start(0), start(1)]

    for k in range(NCHUNK):
        descs[k].wait()
        buf = bufs[k % 2]

        @plsc.parallel_loop(0, CROWS, unroll=2)
        def _(rr):
            for s in range(NSLICE):
                w = buf[rr, pl.ds(s * 16, 16)]
                for kk in range(4):
                    if kk == 0:
                        v = w & 0xFF
                    elif kk < 3:
                        v = lax.shift_right_logical(w, 8 * kk) & 0xFF
                    else:
                        v = lax.shift_right_logical(w, 24)
                    lv = lvecs[s] if kk == 0 else lvecs[s] + kk
                    plsc.addupdate_scatter(cnt, [v, lv], ones)

        if k + 2 < NCHUNK:
            descs[k + 2] = start(k + 2)

    pltpu.sync_copy(cnt, out_hbm.at[wid])


def _tc_body(cnt_ref, embed_ref, fcw_ref, bias_ref, out_ref):
    ct = jnp.sum(cnt_ref[...], axis=0)
    m = lax.dot_general(ct, embed_ref[...],
                        (((0,), (0,)), ((), ())),
                        preferred_element_type=jnp.float32)
    out = lax.dot_general(m * (1.0 / B), fcw_ref[...],
                          (((1,), (1,)), ((), ())),
                          preferred_element_type=jnp.float32)
    out_ref[...] = out[:L] + bias_ref[...]


def kernel(x, embed_weight, fc_weight, fc_bias):
    xp = lax.bitcast_convert_type(
        x.astype(jnp.int8).reshape(XR, XC, 4), jnp.int32)
    cnt3 = _sc_hist(xp)
    out = pl.pallas_call(
        _tc_body,
        out_shape=jax.ShapeDtypeStruct((L, V), jnp.float32),
    )(cnt3, embed_weight, fc_weight, fc_bias.reshape(1, V))
    return out


# speedup vs baseline: 9.2207x; 9.2207x over previous
import functools

import jax
import jax.numpy as jnp
from jax import lax
from jax.experimental import pallas as pl
from jax.experimental.pallas import tpu as pltpu
from jax.experimental.pallas import tpu_sc as plsc

B = 16384
L = 200
V = 17
D = 16
LP = 256
NC, NS = 2, 16
NW = NC * NS
ROWS = B // NW
CROWS = 128
NCHUNK = ROWS // CROWS
NSLICE = 13

_mesh = plsc.VectorSubcoreMesh(core_axis_name="c", subcore_axis_name="s")


@functools.partial(
    pl.kernel,
    out_type=jax.ShapeDtypeStruct((NW, V * LP), jnp.float32),
    mesh=_mesh,
    compiler_params=pltpu.CompilerParams(needs_layout_passes=False),
    scratch_types=[
        pltpu.VMEM((CROWS, L), jnp.int32),
        pltpu.VMEM((CROWS, L), jnp.int32),
        pltpu.VMEM((V * LP,), jnp.float32),
        pltpu.SemaphoreType.DMA,
        pltpu.SemaphoreType.DMA,
    ],
)
def _sc_hist(x_hbm, out_hbm, xb0, xb1, cnt, sem0, sem1):
    wid = lax.axis_index("s") * NC + lax.axis_index("c")
    bufs = (xb0, xb1)
    sems = (sem0, sem1)

    @plsc.parallel_loop(0, V * LP // 16, unroll=4)
    def _(j):
        cnt[pl.ds(j * 16, 16)] = jnp.zeros((16,), jnp.float32)

    row0 = wid * ROWS

    def start(k):
        return pltpu.async_copy(
            x_hbm.at[pl.ds(row0 + k * CROWS, CROWS)], bufs[k % 2], sems[k % 2])

    ones = jnp.ones((16,), jnp.float32)
    iota = lax.iota(jnp.int32, 16)
    tail_mask = iota >= 8
    lvecs = [iota + (c * 16 if c < NSLICE - 1 else L - 16)
             for c in range(NSLICE)]

    descs = [start(0), start(1), None, None]

    for k in range(NCHUNK):
        descs[k].wait()
        buf = bufs[k % 2]

        @plsc.parallel_loop(0, CROWS, unroll=2)
        def _(r):
            for c in range(NSLICE):
                off = c * 16 if c < NSLICE - 1 else L - 16
                v = buf[r, pl.ds(off, 16)]
                idx = lax.shift_left(v, 8) | lvecs[c]
                if c < NSLICE - 1:
                    plsc.addupdate_scatter(cnt, [idx], ones)
                else:
                    plsc.addupdate_scatter(cnt, [idx], ones, mask=tail_mask)

        if k + 2 < NCHUNK:
            descs[k + 2] = start(k + 2)

    pltpu.sync_copy(cnt, out_hbm.at[wid])


def _tc_body(cnt_ref, embed_ref, fcw_ref, bias_ref, out_ref):
    ct = jnp.sum(cnt_ref[...], axis=0)
    m = lax.dot_general(ct, embed_ref[...],
                        (((0,), (0,)), ((), ())),
                        preferred_element_type=jnp.float32)
    out = lax.dot_general(m * (1.0 / B), fcw_ref[...],
                          (((1,), (1,)), ((), ())),
                          preferred_element_type=jnp.float32)
    out_ref[...] = out[:L] + bias_ref[...]


def kernel(x, embed_weight, fc_weight, fc_bias):
    counts = _sc_hist(x.astype(jnp.int32))
    cnt3 = counts.reshape(NW, V, LP)
    out = pl.pallas_call(
        _tc_body,
        out_shape=jax.ShapeDtypeStruct((L, V), jnp.float32),
    )(cnt3, embed_weight, fc_weight, fc_bias.reshape(1, V))
    return out


# speedup vs baseline: 9.5756x; 1.0385x over previous
import functools

import jax
import jax.numpy as jnp
from jax import lax
from jax.experimental import pallas as pl
from jax.experimental.pallas import tpu as pltpu
from jax.experimental.pallas import tpu_sc as plsc

B = 16384
L = 200
V = 17
D = 16
LP = 256
NC, NS = 2, 16
NW = NC * NS
ROWS = B // NW
CROWS = 128
NCHUNK = ROWS // CROWS
NSLICE = 13

_mesh = plsc.VectorSubcoreMesh(core_axis_name="c", subcore_axis_name="s")


@functools.partial(
    pl.kernel,
    out_type=jax.ShapeDtypeStruct((NW, V * LP), jnp.float32),
    mesh=_mesh,
    compiler_params=pltpu.CompilerParams(needs_layout_passes=False),
    scratch_types=[
        pltpu.VMEM((CROWS, L), jnp.int32),
        pltpu.VMEM((CROWS, L), jnp.int32),
        pltpu.VMEM((V * LP,), jnp.float32),
        pltpu.SemaphoreType.DMA,
        pltpu.SemaphoreType.DMA,
    ],
)
def _sc_hist(x_hbm, out_hbm, xb0, xb1, cnt, sem0, sem1):
    wid = lax.axis_index("s") * NC + lax.axis_index("c")
    bufs = (xb0, xb1)
    sems = (sem0, sem1)

    @plsc.parallel_loop(0, V * LP // 16, unroll=4)
    def _(j):
        cnt[pl.ds(j * 16, 16)] = jnp.zeros((16,), jnp.float32)

    row0 = wid * ROWS

    def start(k):
        return pltpu.async_copy(
            x_hbm.at[pl.ds(row0 + k * CROWS, CROWS)], bufs[k % 2], sems[k % 2])

    ones = jnp.ones((16,), jnp.float32)
    iota = lax.iota(jnp.int32, 16)
    tail_mask = iota >= 8
    lvecs = [iota + (c * 16 if c < NSLICE - 1 else L - 16)
             for c in range(NSLICE)]

    descs = [start(0), start(1), None, None]

    for k in range(NCHUNK):
        descs[k].wait()
        buf = bufs[k % 2]

        @plsc.parallel_loop(0, CROWS, unroll=4)
        def _(r):
            for c in range(NSLICE):
                off = c * 16 if c < NSLICE - 1 else L - 16
                v = buf[r, pl.ds(off, 16)]
                idx = lax.shift_left(v, 8) | lvecs[c]
                if c < NSLICE - 1:
                    plsc.addupdate_scatter(cnt, [idx], ones)
                else:
                    plsc.addupdate_scatter(cnt, [idx], ones, mask=tail_mask)

        if k + 2 < NCHUNK:
            descs[k + 2] = start(k + 2)

    pltpu.sync_copy(cnt, out_hbm.at[wid])


def _tc_body(cnt_ref, embed_ref, fcw_ref, bias_ref, out_ref):
    ct = jnp.sum(cnt_ref[...], axis=0).reshape(V, LP)
    m = lax.dot_general(ct, embed_ref[...],
                        (((0,), (0,)), ((), ())),
                        preferred_element_type=jnp.float32)
    out = lax.dot_general(m * (1.0 / B), fcw_ref[...],
                          (((1,), (1,)), ((), ())),
                          preferred_element_type=jnp.float32)
    out_ref[...] = out[:L] + bias_ref[...]


def kernel(x, embed_weight, fc_weight, fc_bias):
    counts = _sc_hist(x.astype(jnp.int32))
    out = pl.pallas_call(
        _tc_body,
        out_shape=jax.ShapeDtypeStruct((L, V), jnp.float32),
    )(counts, embed_weight, fc_weight, fc_bias.reshape(1, V))
    return out
